# Initial kernel scaffold; baseline (speedup 1.0000x reference)
import functools

import jax
import jax.numpy as jnp
from jax import lax
from jax.experimental import pallas as pl
from jax.experimental.pallas import tpu as pltpu
from jax.experimental.pallas import tpu_sc as plsc

N = 10000
NP = 10240
E0 = 160000
EP = 163840
PADE = EP - E0
D = 512
C = 4
CW = 128
BM = 2000
EB = 128
NBLK = EP // 16 // EB
TILE_E = EP // 16
NPT = NP // 16
CPT = N // 16
NPW = NP // 32
EPS = 0.001
DECAY = [(1.0 - EPS) ** t for t in range(3)]

_MESH = plsc.VectorSubcoreMesh(core_axis_name="c", subcore_axis_name="s",
                               num_cores=2, num_subcores=16)


def _rsqrt16(x):
    i = plsc.bitcast(x, jnp.int32)
    i = jnp.int32(0x5F3759DF) - (i >> 1)
    y = plsc.bitcast(i, jnp.float32)
    for _ in range(3):
        y = y * (1.5 - 0.5 * x * y * y)
    return y


def _pre_body(srcp, dstp, dst3, zn, ewr, ews, pT, cT,
              sfv, dfv, d2v, dloc, wn, qv, ones_v, degsl, qloc, ewt,
              pbuf, cbuf, ebuf, deg_s, q_s, dinv_s):
    cid = lax.axis_index("c")
    sid = lax.axis_index("s")
    ebase = sid * TILE_E
    nb640 = sid * NPT

    pltpu.sync_copy(srcp.at[pl.ds(ebase, TILE_E)], sfv)
    pltpu.sync_copy(dstp.at[pl.ds(ebase, TILE_E)], dfv)
    pltpu.sync_copy(dst3.at[pl.ds(sid * NBLK, NBLK)], d2v)
    pltpu.sync_copy(zn.at[pl.ds(nb640, NPT)], deg_s.at[pl.ds(nb640, NPT)])
    pltpu.sync_copy(zn.at[pl.ds(nb640, NPT)], q_s.at[pl.ds(nb640, NPT)])
    for k in range(EB // 16):
        ones_v[pl.ds(k * 16, 16)] = jnp.full((16,), 1.0, jnp.float32)
    plsc.subcore_barrier()

    def _pa(j, _):
        pltpu.sync_copy(ones_v, deg_s.at[d2v.at[j]], add=True)
        return 0
    lax.fori_loop(0, NBLK, _pa, 0)
    plsc.subcore_barrier()

    pltpu.sync_copy(deg_s.at[pl.ds(nb640, NPT)], degsl)
    def _pa2(i, _):
        v = degsl[pl.ds(i * 16, 16)] + 1.0
        degsl[pl.ds(i * 16, 16)] = _rsqrt16(v)
        return 0
    lax.fori_loop(0, NPT // 16, _pa2, 0)
    pltpu.sync_copy(degsl, dinv_s.at[pl.ds(nb640, NPT)])
    plsc.subcore_barrier()
    pltpu.sync_copy(dinv_s, dloc)

    def _pb(j, _):
        for k in range(EB // 16):
            off = j * EB + k * 16
            s16 = sfv[pl.ds(off, 16)]
            d16 = dfv[pl.ds(off, 16)]
            sv = plsc.load_gather(dloc, [s16])
            dv = plsc.load_gather(dloc, [d16])
            qv[pl.ds(k * 16, 16)] = sv
            wn[pl.ds(off, 16)] = sv * dv
        pltpu.sync_copy(qv, q_s.at[d2v.at[j]], add=True)
        return 0
    lax.fori_loop(0, NBLK, _pb, 0)
    plsc.subcore_barrier()

    @pl.when(cid == 0)
    def _():
        for t in range(3):
            dk = jnp.float32(DECAY[t])
            def _sc(i, _, dk=dk):
                ewt[pl.ds(i * 16, 16)] = wn[pl.ds(i * 16, 16)] * dk
                return 0
            lax.fori_loop(0, TILE_E // 16, _sc, 0)
            pltpu.sync_copy(ewt, ewr.at[t].at[pl.ds(ebase, TILE_E)])

    w = cid * 16 + sid
    nb = w * NPW
    pltpu.sync_copy(q_s.at[pl.ds(nb, NPW)], qloc)
    for t in range(3):
        dk = jnp.float32(DECAY[t])
        omd = jnp.float32(1.0 - DECAY[t])
        def _pc(i, _, dk=dk, omd=omd):
            d0 = dloc[pl.ds(nb + i * 16, 16)]
            q16 = qloc[pl.ds(i * 16, 16)]
            sw = d0 * (q16 + d0)
            degt = dk * sw + omd
            dit = _rsqrt16(degt)
            p16 = dit * d0
            s16 = dit * dit * (dk * d0 * d0 + omd)
            pbuf[pl.ds(i * 16, 16)] = p16
            cbuf[pl.ds(i * 16, 16)] = s16 / p16
            ebuf[pl.ds(i * 16, 16)] = dk * d0 * d0 + omd
            return 0
        lax.fori_loop(0, NPW // 16, _pc, 0)
        pltpu.sync_copy(pbuf, pT.at[t].at[pl.ds(nb, NPW)])
        pltpu.sync_copy(cbuf, cT.at[t].at[pl.ds(nb, NPW)])
        pltpu.sync_copy(ebuf, ews.at[t].at[pl.ds(nb, NPW)])


def _precompute(srcp, dstp, dst3, zeros_n):
    f32 = jnp.float32
    kfn = pl.kernel(
        _pre_body,
        out_type=(
            jax.ShapeDtypeStruct((3, EP), f32),
            jax.ShapeDtypeStruct((3, NP), f32),
            jax.ShapeDtypeStruct((3, NP), f32),
            jax.ShapeDtypeStruct((3, NP), f32),
        ),
        mesh=_MESH,
        scratch_types=[
            pltpu.VMEM((TILE_E,), jnp.int32),
            pltpu.VMEM((TILE_E,), jnp.int32),
            pltpu.VMEM((NBLK, EB), jnp.int32),
            pltpu.VMEM((NP,), f32),
            pltpu.VMEM((TILE_E,), f32),
            pltpu.VMEM((EB,), f32),
            pltpu.VMEM((EB,), f32),
            pltpu.VMEM((NPT,), f32),
            pltpu.VMEM((NPW,), f32),
            pltpu.VMEM((TILE_E,), f32),
            pltpu.VMEM((NPW,), f32),
            pltpu.VMEM((NPW,), f32),
            pltpu.VMEM((NPW,), f32),
            pltpu.VMEM_SHARED((NP,), f32),
            pltpu.VMEM_SHARED((NP,), f32),
            pltpu.VMEM_SHARED((NP,), f32),
        ],
    )
    return kfn(srcp, dstp, dst3, zeros_n)


def _spmm_body(hpc, srcp, dst3, zslab, aggc,
               sfv, d2v, r0, r1, sem0, sem1, slab):
    cid = lax.axis_index("c")
    sid = lax.axis_index("s")
    ebase = sid * TILE_E

    pltpu.sync_copy(srcp.at[pl.ds(ebase, TILE_E)], sfv)
    pltpu.sync_copy(dst3.at[pl.ds(sid * NBLK, NBLK)], d2v)

    for j in range(C // 2):
        cidx = cid * (C // 2) + j
        tbl = hpc.at[cidx]
        pltpu.sync_copy(zslab.at[pl.ds(sid * NPT, NPT)],
                        slab.at[pl.ds(sid * NPT, NPT)])
        plsc.subcore_barrier()

        pltpu.async_copy(tbl.at[sfv.at[pl.ds(0, EB)]], r0, sem0)

        def _blk(k, _):
            j0 = 2 * k
            j1 = 2 * k + 1
            pltpu.make_async_copy(tbl.at[sfv.at[pl.ds(0, EB)]], r0, sem0).wait()
            pltpu.async_copy(tbl.at[sfv.at[pl.ds(j1 * EB, EB)]], r1, sem1)
            pltpu.sync_copy(r0, slab.at[d2v.at[j0]], add=True)
            pltpu.make_async_copy(tbl.at[sfv.at[pl.ds(0, EB)]], r1, sem1).wait()
            @pl.when(j1 + 1 < NBLK)
            def _():
                pltpu.async_copy(tbl.at[sfv.at[pl.ds((j1 + 1) * EB, EB)]],
                                 r0, sem0)
            pltpu.sync_copy(r1, slab.at[d2v.at[j1]], add=True)
            return 0
        lax.fori_loop(0, NBLK // 2, _blk, 0)
        plsc.subcore_barrier()
        pltpu.sync_copy(slab.at[pl.ds(sid * CPT, CPT)],
                        aggc.at[cidx].at[pl.ds(sid * CPT, CPT)])
        plsc.subcore_barrier()


def _spmm(hpc, srcp, dst3, zslab):
    f32 = jnp.float32
    kfn = pl.kernel(
        _spmm_body,
        out_type=jax.ShapeDtypeStruct((C, N, CW), f32),
        mesh=_MESH,
        scratch_types=[
            pltpu.VMEM((TILE_E,), jnp.int32),
            pltpu.VMEM((NBLK, EB), jnp.int32),
            pltpu.VMEM((EB, CW), f32),
            pltpu.VMEM((EB, CW), f32),
            pltpu.SemaphoreType.DMA,
            pltpu.SemaphoreType.DMA,
            pltpu.VMEM_SHARED((NP, CW), f32),
        ],
    )
    return kfn(hpc, srcp, dst3, zslab)


def _mm_body(z_ref, w_ref, p_ref, o_ref):
    acc = jnp.dot(z_ref[...], w_ref[...], preferred_element_type=jnp.float32)
    o_ref[0] = p_ref[...] * acc


def _mm(z, W, p2d):
    K = z.shape[1]
    return pl.pallas_call(
        _mm_body,
        grid=(N // BM, C),
        in_specs=[
            pl.BlockSpec((BM, K), lambda m, n: (m, 0)),
            pl.BlockSpec((K, CW), lambda m, n: (0, n)),
            pl.BlockSpec((BM, 1), lambda m, n: (m, 0)),
        ],
        out_specs=pl.BlockSpec((1, BM, CW), lambda m, n: (n, m, 0)),
        out_shape=jax.ShapeDtypeStruct((C, N, CW), jnp.float32),
    )(z, W, p2d)


def _comb_body(decay, agg_ref, hp_ref, p_ref, c_ref, b_ref, a_ref, o_ref):
    u = decay * p_ref[...] * agg_ref[0] + c_ref[...] * hp_ref[0] + b_ref[...]
    o_ref[...] = jnp.maximum(u, 0.0) + a_ref[...] * jnp.minimum(u, 0.0)


def _combine(aggc, hpc, p2d, c2d, b4, a4, decay):
    return pl.pallas_call(
        functools.partial(_comb_body, jnp.float32(decay)),
        grid=(N // BM, C),
        in_specs=[
            pl.BlockSpec((1, BM, CW), lambda m, k: (k, m, 0)),
            pl.BlockSpec((1, BM, CW), lambda m, k: (k, m, 0)),
            pl.BlockSpec((BM, 1), lambda m, k: (m, 0)),
            pl.BlockSpec((BM, 1), lambda m, k: (m, 0)),
            pl.BlockSpec((1, CW), lambda m, k: (k, 0)),
            pl.BlockSpec((1, CW), lambda m, k: (k, 0)),
        ],
        out_specs=pl.BlockSpec((BM, CW), lambda m, k: (m, k)),
        out_shape=jax.ShapeDtypeStruct((N, D), jnp.float32),
    )(aggc, hpc, p2d, c2d, b4, a4)


def kernel(x, edge_index, W0, b0, W1, b1, W2, b2, prelu_a):
    f32 = jnp.float32
    i32 = jnp.int32
    src = edge_index[0].astype(i32)
    dst = edge_index[1].astype(i32)
    pad = jnp.arange(PADE, dtype=i32)
    srcp = jnp.concatenate([src, pad % N])
    dstp = jnp.concatenate([dst, N + pad % (NP - N)])
    dst3 = dstp.reshape(16 * NBLK, EB)
    zeros_n = jnp.zeros((NP,), f32)
    zslab = jnp.zeros((NP, CW), f32)

    ewr, ews, pT, cT = _precompute(srcp, dstp, dst3, zeros_n)

    ew_T = jnp.concatenate([ewr[:, :E0], ews[:, :N]], axis=1)
    loop = jnp.arange(N, dtype=edge_index.dtype)
    ei = jnp.concatenate([edge_index, jnp.stack([loop, loop])], axis=1)
    ei_T = jnp.broadcast_to(ei[None], (3,) + ei.shape)

    params = [(W0, b0.reshape(C, CW)), (W1, b1.reshape(C, CW)),
              (W2, b2.reshape(C, CW))]
    a4 = prelu_a.reshape(C, CW)
    feats = []
    for t in range(3):
        p2d = pT[t, :N, None]
        c2d = cT[t, :N, None]
        z = x
        for (W, b4t) in params:
            hpc = _mm(z, W, p2d)
            aggc = _spmm(hpc, srcp, dst3, zslab)
            z = _combine(aggc, hpc, p2d, c2d, b4t, a4, DECAY[t])
        feats.append(z)
    features_T = jnp.stack(feats)
    return features_T, ei_T, ew_T


# speedup vs baseline: 8.9999x; 8.9999x over previous
import functools

import jax
import jax.numpy as jnp
from jax import lax
from jax.experimental import pallas as pl
from jax.experimental.pallas import tpu as pltpu
from jax.experimental.pallas import tpu_sc as plsc

N = 10000
NP = 10240
E0 = 160000
EP = 163840
PADE = EP - E0
D = 512
C = 4
CW = 128
BM = 2000
EB = 80
NBLK = EP // 16 // EB
TILE_E = EP // 16
NPT = NP // 16
CPT = N // 16
NPW = NP // 32
EPS = 0.001
DECAY = [(1.0 - EPS) ** t for t in range(3)]

_MESH = plsc.VectorSubcoreMesh(core_axis_name="c", subcore_axis_name="s",
                               num_cores=2, num_subcores=16)


def _rsqrt16(x):
    i = lax.bitcast_convert_type(x, jnp.int32)
    i = jnp.int32(0x5F3759DF) - (i >> 1)
    y = lax.bitcast_convert_type(i, jnp.float32)
    for _ in range(3):
        y = y * (1.5 - 0.5 * x * y * y)
    return y


def _pre_body(srcp, dstp, zn, ewr, ews, pT, cT,
              sfv, dfv, d2v, dloc, wn, qv, ones_v, degsl, qloc, ewt,
              pbuf, cbuf, ebuf, deg_s, q_s, dinv_s):
    cid = lax.axis_index("c")
    sid = lax.axis_index("s")
    ebase = sid * TILE_E
    nb640 = sid * NPT

    pltpu.sync_copy(srcp.at[pl.ds(ebase, TILE_E)], sfv)
    pltpu.sync_copy(dstp.at[pl.ds(ebase, TILE_E)], dfv)
    def _mkrows(j, _):
        pltpu.sync_copy(dstp.at[pl.ds(ebase + j * EB, EB)], d2v.at[j])
        return 0
    lax.fori_loop(0, NBLK, _mkrows, 0)
    pltpu.sync_copy(zn.at[pl.ds(nb640, NPT)], deg_s.at[pl.ds(nb640, NPT)])
    pltpu.sync_copy(zn.at[pl.ds(nb640, NPT)], q_s.at[pl.ds(nb640, NPT)])
    for k in range(EB // 16):
        ones_v[pl.ds(k * 16, 16)] = jnp.full((16,), 1.0, jnp.float32)
    plsc.subcore_barrier()

    def _pa(j, _):
        pltpu.sync_copy(ones_v, deg_s.at[d2v.at[j]], add=True)
        return 0
    lax.fori_loop(0, NBLK, _pa, 0)
    plsc.subcore_barrier()

    pltpu.sync_copy(deg_s.at[pl.ds(nb640, NPT)], degsl)
    def _pa2(i, _):
        v = degsl[pl.ds(i * 16, 16)] + 1.0
        degsl[pl.ds(i * 16, 16)] = _rsqrt16(v)
        return 0
    lax.fori_loop(0, NPT // 16, _pa2, 0)
    pltpu.sync_copy(degsl, dinv_s.at[pl.ds(nb640, NPT)])
    plsc.subcore_barrier()
    pltpu.sync_copy(dinv_s, dloc)

    def _pb(j, _):
        for k in range(EB // 16):
            off = j * EB + k * 16
            s16 = sfv[pl.ds(off, 16)]
            d16 = dfv[pl.ds(off, 16)]
            sv = plsc.load_gather(dloc, [s16])
            dv = plsc.load_gather(dloc, [d16])
            qv[pl.ds(k * 16, 16)] = sv
            wn[pl.ds(off, 16)] = sv * dv
        pltpu.sync_copy(qv, q_s.at[d2v.at[j]], add=True)
        return 0
    lax.fori_loop(0, NBLK, _pb, 0)
    plsc.subcore_barrier()

    @pl.when(cid == 0)
    def _():
        for t in range(3):
            dk = jnp.float32(DECAY[t])
            def _sc(i, _, dk=dk):
                ewt[pl.ds(i * 16, 16)] = wn[pl.ds(i * 16, 16)] * dk
                return 0
            lax.fori_loop(0, TILE_E // 16, _sc, 0)
            pltpu.sync_copy(ewt, ewr.at[pl.ds(t * EP + ebase, TILE_E)])

    w = cid * 16 + sid
    nb = w * NPW
    pltpu.sync_copy(q_s.at[pl.ds(nb, NPW)], qloc)
    for t in range(3):
        dk = jnp.float32(DECAY[t])
        omd = jnp.float32(1.0 - DECAY[t])
        def _pc(i, _, dk=dk, omd=omd):
            d0 = dloc[pl.ds(nb + i * 16, 16)]
            q16 = qloc[pl.ds(i * 16, 16)]
            sw = d0 * (q16 + d0)
            degt = dk * sw + omd
            dit = _rsqrt16(degt)
            p16 = dit * d0
            s16 = dit * dit * (dk * d0 * d0 + omd)
            pbuf[pl.ds(i * 16, 16)] = p16
            cbuf[pl.ds(i * 16, 16)] = s16 / p16
            ebuf[pl.ds(i * 16, 16)] = dk * d0 * d0 + omd
            return 0
        lax.fori_loop(0, NPW // 16, _pc, 0)
        pltpu.sync_copy(pbuf, pT.at[pl.ds(t * NP + nb, NPW)])
        pltpu.sync_copy(cbuf, cT.at[pl.ds(t * NP + nb, NPW)])
        pltpu.sync_copy(ebuf, ews.at[pl.ds(t * NP + nb, NPW)])


def _precompute(srcp, dstp, zeros_n):
    f32 = jnp.float32
    kfn = pl.kernel(
        _pre_body,
        out_type=(
            jax.ShapeDtypeStruct((3 * EP,), f32),
            jax.ShapeDtypeStruct((3 * NP,), f32),
            jax.ShapeDtypeStruct((3 * NP,), f32),
            jax.ShapeDtypeStruct((3 * NP,), f32),
        ),
        mesh=_MESH,
        scratch_types=[
            pltpu.VMEM((TILE_E,), jnp.int32),
            pltpu.VMEM((TILE_E,), jnp.int32),
            pltpu.VMEM((NBLK, EB), jnp.int32),
            pltpu.VMEM((NP,), f32),
            pltpu.VMEM((TILE_E,), f32),
            pltpu.VMEM((EB,), f32),
            pltpu.VMEM((EB,), f32),
            pltpu.VMEM((NPT,), f32),
            pltpu.VMEM((NPW,), f32),
            pltpu.VMEM((TILE_E,), f32),
            pltpu.VMEM((NPW,), f32),
            pltpu.VMEM((NPW,), f32),
            pltpu.VMEM((NPW,), f32),
            pltpu.VMEM_SHARED((NP,), f32),
            pltpu.VMEM_SHARED((NP,), f32),
            pltpu.VMEM_SHARED((NP,), f32),
        ],
        compiler_params=pltpu.CompilerParams(needs_layout_passes=False),
    )
    return kfn(srcp, dstp, zeros_n)


def _spmm_body(hpc, srcp, dstp, zslab, aggc,
               sfv, d2v, r0, r1, sem0, sem1, slab):
    cid = lax.axis_index("c")
    sid = lax.axis_index("s")
    ebase = sid * TILE_E

    pltpu.sync_copy(srcp.at[pl.ds(ebase, TILE_E)], sfv)
    def _mkrows(j, _):
        pltpu.sync_copy(dstp.at[pl.ds(ebase + j * EB, EB)], d2v.at[j])
        return 0
    lax.fori_loop(0, NBLK, _mkrows, 0)

    for j in range(C // 2):
        cidx = cid * (C // 2) + j
        tbl = hpc.at[cidx]
        pltpu.sync_copy(zslab.at[pl.ds(sid * NPT, NPT)],
                        slab.at[pl.ds(sid * NPT, NPT)])
        plsc.subcore_barrier()

        pltpu.async_copy(tbl.at[sfv.at[pl.ds(0, EB)]], r0, sem0)

        def _blk(k, _):
            j0 = 2 * k
            j1 = 2 * k + 1
            pltpu.make_async_copy(tbl.at[sfv.at[pl.ds(0, EB)]], r0, sem0).wait()
            pltpu.async_copy(tbl.at[sfv.at[pl.ds(j1 * EB, EB)]], r1, sem1)
            pltpu.sync_copy(r0, slab.at[d2v.at[j0]], add=True)
            pltpu.make_async_copy(tbl.at[sfv.at[pl.ds(0, EB)]], r1, sem1).wait()
            @pl.when(j1 + 1 < NBLK)
            def _():
                pltpu.async_copy(tbl.at[sfv.at[pl.ds((j1 + 1) * EB, EB)]],
                                 r0, sem0)
            pltpu.sync_copy(r1, slab.at[d2v.at[j1]], add=True)
            return 0
        lax.fori_loop(0, NBLK // 2, _blk, 0)
        plsc.subcore_barrier()
        pltpu.sync_copy(slab.at[pl.ds(sid * NPT, NPT)],
                        aggc.at[cidx].at[pl.ds(sid * NPT, NPT)])
        plsc.subcore_barrier()


def _spmm(hpc, srcp, dstp, zslab):
    f32 = jnp.float32
    kfn = pl.kernel(
        _spmm_body,
        out_type=jax.ShapeDtypeStruct((C, NP, CW), f32),
        mesh=_MESH,
        scratch_types=[
            pltpu.VMEM((TILE_E,), jnp.int32),
            pltpu.VMEM((NBLK, EB), jnp.int32),
            pltpu.VMEM((EB, CW), f32),
            pltpu.VMEM((EB, CW), f32),
            pltpu.SemaphoreType.DMA,
            pltpu.SemaphoreType.DMA,
            pltpu.VMEM_SHARED((NP, CW), f32),
        ],
        compiler_params=pltpu.CompilerParams(needs_layout_passes=False),
    )
    return kfn(hpc, srcp, dstp, zslab)


def _mm_body(z_ref, w_ref, p_ref, o_ref):
    h = jnp.dot(z_ref[...], w_ref[...], preferred_element_type=jnp.float32)
    hp = p_ref[...] * h
    for kk in range(C):
        o_ref[kk] = hp[:, kk * CW:(kk + 1) * CW]


def _mm(z, W, p2d):
    K = z.shape[1]
    return pl.pallas_call(
        _mm_body,
        grid=(N // BM,),
        in_specs=[
            pl.BlockSpec((BM, K), lambda m: (m, 0)),
            pl.BlockSpec((K, D), lambda m: (0, 0)),
            pl.BlockSpec((BM, 1), lambda m: (m, 0)),
        ],
        out_specs=pl.BlockSpec((C, BM, CW), lambda m: (0, m, 0)),
        out_shape=jax.ShapeDtypeStruct((C, N, CW), jnp.float32),
    )(z, W, p2d)


def _comb_body(decay, agg_ref, hp_ref, p_ref, c_ref, b_ref, a_ref, o_ref):
    pv = p_ref[...]
    cv = c_ref[...]
    for kk in range(C):
        u = (decay * pv * agg_ref[kk] + cv * hp_ref[kk]
             + b_ref[0, :, kk * CW:(kk + 1) * CW])
        o_ref[:, kk * CW:(kk + 1) * CW] = (
            jnp.maximum(u, 0.0)
            + a_ref[0, :, kk * CW:(kk + 1) * CW] * jnp.minimum(u, 0.0))


def _combine(aggc, hpc, p2d, c2d, b2, a2, decay):
    return pl.pallas_call(
        functools.partial(_comb_body, float(decay)),
        grid=(N // BM,),
        in_specs=[
            pl.BlockSpec((C, BM, CW), lambda m: (0, m, 0)),
            pl.BlockSpec((C, BM, CW), lambda m: (0, m, 0)),
            pl.BlockSpec((BM, 1), lambda m: (m, 0)),
            pl.BlockSpec((BM, 1), lambda m: (m, 0)),
            pl.BlockSpec((1, 1, D), lambda m: (0, 0, 0)),
            pl.BlockSpec((1, 1, D), lambda m: (0, 0, 0)),
        ],
        out_specs=pl.BlockSpec((BM, D), lambda m: (m, 0)),
        out_shape=jax.ShapeDtypeStruct((N, D), jnp.float32),
    )(aggc, hpc, p2d, c2d, b2, a2)


def kernel(x, edge_index, W0, b0, W1, b1, W2, b2, prelu_a):
    f32 = jnp.float32
    i32 = jnp.int32
    src = edge_index[0].astype(i32)
    dst = edge_index[1].astype(i32)
    pad = jnp.arange(PADE, dtype=i32)
    srcp = jnp.concatenate([src, pad % N])
    dstp = jnp.concatenate([dst, N + pad % (NP - N)])
    zeros_n = jnp.zeros((NP,), f32)
    zslab = jnp.zeros((NP, CW), f32)

    ewr, ews, pT, cT = _precompute(srcp, dstp, zeros_n)
    ewr = ewr.reshape(3, EP)
    ews = ews.reshape(3, NP)
    pT = pT.reshape(3, NP)
    cT = cT.reshape(3, NP)

    ew_T = jnp.concatenate([ewr[:, :E0], ews[:, :N]], axis=1)
    loop = jnp.arange(N, dtype=edge_index.dtype)
    ei = jnp.concatenate([edge_index, jnp.stack([loop, loop])], axis=1)
    ei_T = jnp.broadcast_to(ei[None], (3,) + ei.shape)

    params = [(W0, b0.reshape(1, 1, D)), (W1, b1.reshape(1, 1, D)),
              (W2, b2.reshape(1, 1, D))]
    a4 = prelu_a.reshape(1, 1, D)
    feats = []
    for t in range(3):
        p2d = pT[t, :N, None]
        c2d = cT[t, :N, None]
        z = x
        for (W, b4t) in params:
            hpc = _mm(z, W, p2d)
            aggc = _spmm(hpc, srcp, dstp, zslab)
            z = _combine(aggc, hpc, p2d, c2d, b4t, a4, DECAY[t])
        feats.append(z)
    features_T = jnp.stack(feats)
    return features_T, ei_T, ew_T


# speedup vs baseline: 9.0845x; 1.0094x over previous
import functools

import jax
import jax.numpy as jnp
from jax import lax
from jax.experimental import pallas as pl
from jax.experimental.pallas import tpu as pltpu
from jax.experimental.pallas import tpu_sc as plsc

N = 10000
NP = 10240
E0 = 160000
EP = 163840
PADE = EP - E0
D = 512
C = 4
CW = 128
BM = 2000
EB = 80
NBLK = EP // 16 // EB
TILE_E = EP // 16
NPT = NP // 16
CPT = N // 16
NPW = NP // 32
EPS = 0.001
DECAY = [(1.0 - EPS) ** t for t in range(3)]

_MESH = plsc.VectorSubcoreMesh(core_axis_name="c", subcore_axis_name="s",
                               num_cores=2, num_subcores=16)


def _rsqrt16(x):
    i = lax.bitcast_convert_type(x, jnp.int32)
    i = jnp.int32(0x5F3759DF) - (i >> 1)
    y = lax.bitcast_convert_type(i, jnp.float32)
    for _ in range(3):
        y = y * (1.5 - 0.5 * x * y * y)
    return y


def _pre_body(srcp, dstp, zn, ewr, ews, pT, cT,
              sfv, dfv, d2v, dloc, wn, qv, ones_v, degsl, qloc, ewt,
              pbuf, cbuf, ebuf, deg_s, q_s, dinv_s):
    cid = lax.axis_index("c")
    sid = lax.axis_index("s")
    ebase = sid * TILE_E
    nb640 = sid * NPT

    pltpu.sync_copy(srcp.at[pl.ds(ebase, TILE_E)], sfv)
    pltpu.sync_copy(dstp.at[pl.ds(ebase, TILE_E)], dfv)
    def _mkrows(j, _):
        pltpu.sync_copy(dstp.at[pl.ds(ebase + j * EB, EB)], d2v.at[j])
        return 0
    lax.fori_loop(0, NBLK, _mkrows, 0)
    pltpu.sync_copy(zn.at[pl.ds(nb640, NPT)], deg_s.at[pl.ds(nb640, NPT)])
    pltpu.sync_copy(zn.at[pl.ds(nb640, NPT)], q_s.at[pl.ds(nb640, NPT)])
    for k in range(EB // 16):
        ones_v[pl.ds(k * 16, 16)] = jnp.full((16,), 1.0, jnp.float32)
    plsc.subcore_barrier()

    def _pa(j, _):
        pltpu.sync_copy(ones_v, deg_s.at[d2v.at[j]], add=True)
        return 0
    lax.fori_loop(0, NBLK, _pa, 0)
    plsc.subcore_barrier()

    pltpu.sync_copy(deg_s.at[pl.ds(nb640, NPT)], degsl)
    def _pa2(i, _):
        v = degsl[pl.ds(i * 16, 16)] + 1.0
        degsl[pl.ds(i * 16, 16)] = _rsqrt16(v)
        return 0
    lax.fori_loop(0, NPT // 16, _pa2, 0)
    pltpu.sync_copy(degsl, dinv_s.at[pl.ds(nb640, NPT)])
    plsc.subcore_barrier()
    pltpu.sync_copy(dinv_s, dloc)

    def _pb(j, _):
        for k in range(EB // 16):
            off = j * EB + k * 16
            s16 = sfv[pl.ds(off, 16)]
            d16 = dfv[pl.ds(off, 16)]
            sv = plsc.load_gather(dloc, [s16])
            dv = plsc.load_gather(dloc, [d16])
            qv[pl.ds(k * 16, 16)] = sv
            wn[pl.ds(off, 16)] = sv * dv
        pltpu.sync_copy(qv, q_s.at[d2v.at[j]], add=True)
        return 0
    lax.fori_loop(0, NBLK, _pb, 0)
    plsc.subcore_barrier()

    @pl.when(cid == 0)
    def _():
        for t in range(3):
            dk = jnp.float32(DECAY[t])
            def _sc(i, _, dk=dk):
                ewt[pl.ds(i * 16, 16)] = wn[pl.ds(i * 16, 16)] * dk
                return 0
            lax.fori_loop(0, TILE_E // 16, _sc, 0)
            pltpu.sync_copy(ewt, ewr.at[pl.ds(t * EP + ebase, TILE_E)])

    w = cid * 16 + sid
    nb = w * NPW
    pltpu.sync_copy(q_s.at[pl.ds(nb, NPW)], qloc)
    for t in range(3):
        dk = jnp.float32(DECAY[t])
        omd = jnp.float32(1.0 - DECAY[t])
        def _pc(i, _, dk=dk, omd=omd):
            d0 = dloc[pl.ds(nb + i * 16, 16)]
            q16 = qloc[pl.ds(i * 16, 16)]
            sw = d0 * (q16 + d0)
            degt = dk * sw + omd
            dit = _rsqrt16(degt)
            p16 = dit * d0
            s16 = dit * dit * (dk * d0 * d0 + omd)
            pbuf[pl.ds(i * 16, 16)] = p16
            cbuf[pl.ds(i * 16, 16)] = s16 / p16
            ebuf[pl.ds(i * 16, 16)] = dk * d0 * d0 + omd
            return 0
        lax.fori_loop(0, NPW // 16, _pc, 0)
        pltpu.sync_copy(pbuf, pT.at[pl.ds(t * NP + nb, NPW)])
        pltpu.sync_copy(cbuf, cT.at[pl.ds(t * NP + nb, NPW)])
        pltpu.sync_copy(ebuf, ews.at[pl.ds(t * NP + nb, NPW)])


def _precompute(srcp, dstp, zeros_n):
    f32 = jnp.float32
    kfn = pl.kernel(
        _pre_body,
        out_type=(
            jax.ShapeDtypeStruct((3 * EP,), f32),
            jax.ShapeDtypeStruct((3 * NP,), f32),
            jax.ShapeDtypeStruct((3 * NP,), f32),
            jax.ShapeDtypeStruct((3 * NP,), f32),
        ),
        mesh=_MESH,
        scratch_types=[
            pltpu.VMEM((TILE_E,), jnp.int32),
            pltpu.VMEM((TILE_E,), jnp.int32),
            pltpu.VMEM((NBLK, EB), jnp.int32),
            pltpu.VMEM((NP,), f32),
            pltpu.VMEM((TILE_E,), f32),
            pltpu.VMEM((EB,), f32),
            pltpu.VMEM((EB,), f32),
            pltpu.VMEM((NPT,), f32),
            pltpu.VMEM((NPW,), f32),
            pltpu.VMEM((TILE_E,), f32),
            pltpu.VMEM((NPW,), f32),
            pltpu.VMEM((NPW,), f32),
            pltpu.VMEM((NPW,), f32),
            pltpu.VMEM_SHARED((NP,), f32),
            pltpu.VMEM_SHARED((NP,), f32),
            pltpu.VMEM_SHARED((NP,), f32),
        ],
        compiler_params=pltpu.CompilerParams(needs_layout_passes=False),
    )
    return kfn(srcp, dstp, zeros_n)


def _spmm_body(hpc, srcp, dstp, zslab, aggc,
               sfv, d2v, r0, r1, sg0, sg1, ss0, ss1, slab):
    cid = lax.axis_index("c")
    sid = lax.axis_index("s")
    ebase = sid * TILE_E

    pltpu.sync_copy(srcp.at[pl.ds(ebase, TILE_E)], sfv)
    def _mkrows(j, _):
        pltpu.sync_copy(dstp.at[pl.ds(ebase + j * EB, EB)], d2v.at[j])
        return 0
    lax.fori_loop(0, NBLK, _mkrows, 0)

    for j in range(C // 2):
        cidx = cid * (C // 2) + j
        tbl = hpc.at[cidx]
        pltpu.sync_copy(zslab.at[pl.ds(sid * NPT, NPT)],
                        slab.at[pl.ds(sid * NPT, NPT)])
        plsc.subcore_barrier()

        pltpu.async_copy(tbl.at[sfv.at[pl.ds(0, EB)]], r0, sg0)
        pltpu.async_copy(tbl.at[sfv.at[pl.ds(EB, EB)]], r1, sg1)

        def _blk(k, _):
            j0 = 2 * k
            j1 = 2 * k + 1
            pltpu.make_async_copy(tbl.at[sfv.at[pl.ds(0, EB)]], r0, sg0).wait()
            pltpu.async_copy(r0, slab.at[d2v.at[j0]], ss0, add=True)
            pltpu.make_async_copy(tbl.at[sfv.at[pl.ds(0, EB)]], r1, sg1).wait()
            pltpu.async_copy(r1, slab.at[d2v.at[j1]], ss1, add=True)
            pltpu.make_async_copy(r0, slab.at[d2v.at[0]], ss0).wait()
            @pl.when(j0 + 2 < NBLK)
            def _():
                pltpu.async_copy(tbl.at[sfv.at[pl.ds((j0 + 2) * EB, EB)]],
                                 r0, sg0)
            pltpu.make_async_copy(r1, slab.at[d2v.at[0]], ss1).wait()
            @pl.when(j1 + 2 < NBLK)
            def _():
                pltpu.async_copy(tbl.at[sfv.at[pl.ds((j1 + 2) * EB, EB)]],
                                 r1, sg1)
            return 0
        lax.fori_loop(0, NBLK // 2, _blk, 0)
        plsc.subcore_barrier()
        pltpu.sync_copy(slab.at[pl.ds(sid * NPT, NPT)],
                        aggc.at[cidx].at[pl.ds(sid * NPT, NPT)])
        plsc.subcore_barrier()


def _spmm(hpc, srcp, dstp, zslab):
    f32 = jnp.float32
    kfn = pl.kernel(
        _spmm_body,
        out_type=jax.ShapeDtypeStruct((C, NP, CW), f32),
        mesh=_MESH,
        scratch_types=[
            pltpu.VMEM((TILE_E,), jnp.int32),
            pltpu.VMEM((NBLK, EB), jnp.int32),
            pltpu.VMEM((EB, CW), f32),
            pltpu.VMEM((EB, CW), f32),
            pltpu.SemaphoreType.DMA,
            pltpu.SemaphoreType.DMA,
            pltpu.SemaphoreType.DMA,
            pltpu.SemaphoreType.DMA,
            pltpu.VMEM_SHARED((NP, CW), f32),
        ],
        compiler_params=pltpu.CompilerParams(needs_layout_passes=False),
    )
    return kfn(hpc, srcp, dstp, zslab)


def _mm_body(z_ref, w_ref, p_ref, o_ref):
    h = jnp.dot(z_ref[...], w_ref[...], preferred_element_type=jnp.float32)
    hp = p_ref[...] * h
    for kk in range(C):
        o_ref[kk] = hp[:, kk * CW:(kk + 1) * CW]


def _mm(z, W, p2d):
    K = z.shape[1]
    return pl.pallas_call(
        _mm_body,
        grid=(N // BM,),
        in_specs=[
            pl.BlockSpec((BM, K), lambda m: (m, 0)),
            pl.BlockSpec((K, D), lambda m: (0, 0)),
            pl.BlockSpec((BM, 1), lambda m: (m, 0)),
        ],
        out_specs=pl.BlockSpec((C, BM, CW), lambda m: (0, m, 0)),
        out_shape=jax.ShapeDtypeStruct((C, N, CW), jnp.float32),
    )(z, W, p2d)


def _comb_body(decay, agg_ref, hp_ref, p_ref, c_ref, b_ref, a_ref, o_ref):
    pv = p_ref[...]
    cv = c_ref[...]
    for kk in range(C):
        u = (decay * pv * agg_ref[kk] + cv * hp_ref[kk]
             + b_ref[0, :, kk * CW:(kk + 1) * CW])
        o_ref[:, kk * CW:(kk + 1) * CW] = (
            jnp.maximum(u, 0.0)
            + a_ref[0, :, kk * CW:(kk + 1) * CW] * jnp.minimum(u, 0.0))


def _combine(aggc, hpc, p2d, c2d, b2, a2, decay):
    return pl.pallas_call(
        functools.partial(_comb_body, float(decay)),
        grid=(N // BM,),
        in_specs=[
            pl.BlockSpec((C, BM, CW), lambda m: (0, m, 0)),
            pl.BlockSpec((C, BM, CW), lambda m: (0, m, 0)),
            pl.BlockSpec((BM, 1), lambda m: (m, 0)),
            pl.BlockSpec((BM, 1), lambda m: (m, 0)),
            pl.BlockSpec((1, 1, D), lambda m: (0, 0, 0)),
            pl.BlockSpec((1, 1, D), lambda m: (0, 0, 0)),
        ],
        out_specs=pl.BlockSpec((BM, D), lambda m: (m, 0)),
        out_shape=jax.ShapeDtypeStruct((N, D), jnp.float32),
    )(aggc, hpc, p2d, c2d, b2, a2)


def kernel(x, edge_index, W0, b0, W1, b1, W2, b2, prelu_a):
    f32 = jnp.float32
    i32 = jnp.int32
    src = edge_index[0].astype(i32)
    dst = edge_index[1].astype(i32)
    pad = jnp.arange(PADE, dtype=i32)
    srcp = jnp.concatenate([src, pad % N])
    dstp = jnp.concatenate([dst, N + pad % (NP - N)])
    zeros_n = jnp.zeros((NP,), f32)
    zslab = jnp.zeros((NP, CW), f32)

    ewr, ews, pT, cT = _precompute(srcp, dstp, zeros_n)
    ewr = ewr.reshape(3, EP)
    ews = ews.reshape(3, NP)
    pT = pT.reshape(3, NP)
    cT = cT.reshape(3, NP)

    ew_T = jnp.concatenate([ewr[:, :E0], ews[:, :N]], axis=1)
    loop = jnp.arange(N, dtype=edge_index.dtype)
    ei = jnp.concatenate([edge_index, jnp.stack([loop, loop])], axis=1)
    ei_T = jnp.broadcast_to(ei[None], (3,) + ei.shape)

    params = [(W0, b0.reshape(1, 1, D)), (W1, b1.reshape(1, 1, D)),
              (W2, b2.reshape(1, 1, D))]
    a4 = prelu_a.reshape(1, 1, D)
    feats = []
    for t in range(3):
        p2d = pT[t, :N, None]
        c2d = cT[t, :N, None]
        z = x
        for (W, b4t) in params:
            hpc = _mm(z, W, p2d)
            aggc = _spmm(hpc, srcp, dstp, zslab)
            z = _combine(aggc, hpc, p2d, c2d, b4t, a4, DECAY[t])
        feats.append(z)
    features_T = jnp.stack(feats)
    return features_T, ei_T, ew_T


# speedup vs baseline: 9.4154x; 1.0364x over previous
import functools

import jax
import jax.numpy as jnp
from jax import lax
from jax.experimental import pallas as pl
from jax.experimental.pallas import tpu as pltpu
from jax.experimental.pallas import tpu_sc as plsc

N = 10000
NP = 10240
E0 = 160000
D = 512
C = 4
CW = 128
BM = 2000
EB = 80
TILE_E = E0 // 16
NBLK = TILE_E // EB
NPT = NP // 16
CPT = N // 16
NPW = NP // 32
EPS = 0.001
DECAY = [(1.0 - EPS) ** t for t in range(3)]

_MESH = plsc.VectorSubcoreMesh(core_axis_name="c", subcore_axis_name="s",
                               num_cores=2, num_subcores=16)


def _rsqrt16(x):
    i = lax.bitcast_convert_type(x, jnp.int32)
    i = jnp.int32(0x5F3759DF) - (i >> 1)
    y = lax.bitcast_convert_type(i, jnp.float32)
    for _ in range(3):
        y = y * (1.5 - 0.5 * x * y * y)
    return y


def _pre_body(srcp, dstp, zn, ewr, ews, pT, cT,
              sfv, dfv, dloc, wn, qv, ones_v, degsl, qloc, ewt,
              pbuf, cbuf, ebuf, deg_s, q_s, dinv_s):
    cid = lax.axis_index("c")
    sid = lax.axis_index("s")
    ebase = sid * TILE_E
    nb640 = sid * NPT

    pltpu.sync_copy(srcp.at[pl.ds(ebase, TILE_E)], sfv)
    pltpu.sync_copy(dstp.at[pl.ds(ebase, TILE_E)], dfv)
    pltpu.sync_copy(zn.at[pl.ds(nb640, NPT)], deg_s.at[pl.ds(nb640, NPT)])
    pltpu.sync_copy(zn.at[pl.ds(nb640, NPT)], q_s.at[pl.ds(nb640, NPT)])
    def _fill1(i, _):
        ones_v[pl.ds(i * 16, 16)] = jnp.full((16,), 1.0, jnp.float32)
        return 0
    lax.fori_loop(0, TILE_E // 16, _fill1, 0)
    plsc.subcore_barrier()

    pltpu.sync_copy(ones_v, deg_s.at[dfv], add=True)
    plsc.subcore_barrier()

    pltpu.sync_copy(deg_s.at[pl.ds(nb640, NPT)], degsl)
    def _pa2(i, _):
        v = degsl[pl.ds(i * 16, 16)] + 1.0
        degsl[pl.ds(i * 16, 16)] = _rsqrt16(v)
        return 0
    lax.fori_loop(0, NPT // 16, _pa2, 0)
    pltpu.sync_copy(degsl, dinv_s.at[pl.ds(nb640, NPT)])
    plsc.subcore_barrier()
    pltpu.sync_copy(dinv_s, dloc)

    def _pb(i, _):
        off = i * 16
        s16 = sfv[pl.ds(off, 16)]
        d16 = dfv[pl.ds(off, 16)]
        sv = plsc.load_gather(dloc, [s16])
        dv = plsc.load_gather(dloc, [d16])
        qv[pl.ds(off, 16)] = sv
        wn[pl.ds(off, 16)] = sv * dv
        return 0
    lax.fori_loop(0, TILE_E // 16, _pb, 0)
    pltpu.sync_copy(qv, q_s.at[dfv], add=True)
    plsc.subcore_barrier()

    @pl.when(cid == 0)
    def _():
        for t in range(3):
            dk = jnp.float32(DECAY[t])
            def _sc(i, _, dk=dk):
                ewt[pl.ds(i * 16, 16)] = wn[pl.ds(i * 16, 16)] * dk
                return 0
            lax.fori_loop(0, TILE_E // 16, _sc, 0)
            pltpu.sync_copy(ewt, ewr.at[pl.ds(t * E0 + ebase, TILE_E)])

    w = cid * 16 + sid
    nb = w * NPW
    pltpu.sync_copy(q_s.at[pl.ds(nb, NPW)], qloc)
    for t in range(3):
        dk = jnp.float32(DECAY[t])
        omd = jnp.float32(1.0 - DECAY[t])
        def _pc(i, _, dk=dk, omd=omd):
            d0 = dloc[pl.ds(nb + i * 16, 16)]
            q16 = qloc[pl.ds(i * 16, 16)]
            sw = d0 * (q16 + d0)
            degt = dk * sw + omd
            dit = _rsqrt16(degt)
            p16 = dit * d0
            s16 = dit * dit * (dk * d0 * d0 + omd)
            pbuf[pl.ds(i * 16, 16)] = p16
            cbuf[pl.ds(i * 16, 16)] = s16 / p16
            ebuf[pl.ds(i * 16, 16)] = dk * d0 * d0 + omd
            return 0
        lax.fori_loop(0, NPW // 16, _pc, 0)
        pltpu.sync_copy(pbuf, pT.at[pl.ds(t * NP + nb, NPW)])
        pltpu.sync_copy(cbuf, cT.at[pl.ds(t * NP + nb, NPW)])
        pltpu.sync_copy(ebuf, ews.at[pl.ds(t * NP + nb, NPW)])


def _precompute(srcp, dstp, zeros_n):
    f32 = jnp.float32
    kfn = pl.kernel(
        _pre_body,
        out_type=(
            jax.ShapeDtypeStruct((3 * E0,), f32),
            jax.ShapeDtypeStruct((3 * NP,), f32),
            jax.ShapeDtypeStruct((3 * NP,), f32),
            jax.ShapeDtypeStruct((3 * NP,), f32),
        ),
        mesh=_MESH,
        scratch_types=[
            pltpu.VMEM((TILE_E,), jnp.int32),
            pltpu.VMEM((TILE_E,), jnp.int32),
            pltpu.VMEM((NP,), f32),
            pltpu.VMEM((TILE_E,), f32),
            pltpu.VMEM((TILE_E,), f32),
            pltpu.VMEM((TILE_E,), f32),
            pltpu.VMEM((NPT,), f32),
            pltpu.VMEM((NPW,), f32),
            pltpu.VMEM((TILE_E,), f32),
            pltpu.VMEM((NPW,), f32),
            pltpu.VMEM((NPW,), f32),
            pltpu.VMEM((NPW,), f32),
            pltpu.VMEM_SHARED((NP,), f32),
            pltpu.VMEM_SHARED((NP,), f32),
            pltpu.VMEM_SHARED((NP,), f32),
        ],
        compiler_params=pltpu.CompilerParams(needs_layout_passes=False),
    )
    return kfn(srcp, dstp, zeros_n)


def _spmm_body(hpc, srcp, dstp, zslab, aggc,
               sfv, d2v, r0, r1, sg0, sg1, ss0, ss1, slab):
    cid = lax.axis_index("c")
    sid = lax.axis_index("s")
    ebase = sid * TILE_E

    pltpu.sync_copy(srcp.at[pl.ds(ebase, TILE_E)], sfv)
    def _mkrows(j, _):
        pltpu.sync_copy(dstp.at[pl.ds(ebase + j * EB, EB)], d2v.at[j])
        return 0
    lax.fori_loop(0, NBLK, _mkrows, 0)

    for j in range(C // 2):
        cidx = cid * (C // 2) + j
        tbl = hpc.at[cidx]
        pltpu.sync_copy(zslab.at[pl.ds(sid * NPT, NPT)],
                        slab.at[pl.ds(sid * NPT, NPT)])
        plsc.subcore_barrier()

        pltpu.async_copy(tbl.at[sfv.at[pl.ds(0, EB)]], r0, sg0)
        pltpu.async_copy(tbl.at[sfv.at[pl.ds(EB, EB)]], r1, sg1)

        def _blk(k, _):
            j0 = 2 * k
            j1 = 2 * k + 1
            pltpu.make_async_copy(tbl.at[sfv.at[pl.ds(0, EB)]], r0, sg0).wait()
            pltpu.async_copy(r0, slab.at[d2v.at[j0]], ss0, add=True)
            pltpu.make_async_copy(tbl.at[sfv.at[pl.ds(0, EB)]], r1, sg1).wait()
            pltpu.async_copy(r1, slab.at[d2v.at[j1]], ss1, add=True)
            pltpu.make_async_copy(r0, slab.at[d2v.at[0]], ss0).wait()
            @pl.when(j0 + 2 < NBLK)
            def _():
                pltpu.async_copy(tbl.at[sfv.at[pl.ds((j0 + 2) * EB, EB)]],
                                 r0, sg0)
            pltpu.make_async_copy(r1, slab.at[d2v.at[0]], ss1).wait()
            @pl.when(j1 + 2 < NBLK)
            def _():
                pltpu.async_copy(tbl.at[sfv.at[pl.ds((j1 + 2) * EB, EB)]],
                                 r1, sg1)
            return 0
        lax.fori_loop(0, NBLK // 2, _blk, 0)
        if NBLK % 2 == 1:
            pltpu.make_async_copy(tbl.at[sfv.at[pl.ds(0, EB)]], r0, sg0).wait()
            pltpu.async_copy(r0, slab.at[d2v.at[NBLK - 1]], ss0, add=True)
            pltpu.make_async_copy(r0, slab.at[d2v.at[0]], ss0).wait()
        plsc.subcore_barrier()
        pltpu.sync_copy(slab.at[pl.ds(sid * NPT, NPT)],
                        aggc.at[cidx].at[pl.ds(sid * NPT, NPT)])
        plsc.subcore_barrier()


def _spmm(hpc, srcp, dstp, zslab):
    f32 = jnp.float32
    kfn = pl.kernel(
        _spmm_body,
        out_type=jax.ShapeDtypeStruct((C, NP, CW), f32),
        mesh=_MESH,
        scratch_types=[
            pltpu.VMEM((TILE_E,), jnp.int32),
            pltpu.VMEM((NBLK, EB), jnp.int32),
            pltpu.VMEM((EB, CW), f32),
            pltpu.VMEM((EB, CW), f32),
            pltpu.SemaphoreType.DMA,
            pltpu.SemaphoreType.DMA,
            pltpu.SemaphoreType.DMA,
            pltpu.SemaphoreType.DMA,
            pltpu.VMEM_SHARED((NP, CW), f32),
        ],
        compiler_params=pltpu.CompilerParams(needs_layout_passes=False),
    )
    return kfn(hpc, srcp, dstp, zslab)


def _mm_body(z_ref, w_ref, p_ref, o_ref):
    h = jnp.dot(z_ref[...], w_ref[...], preferred_element_type=jnp.float32)
    hp = p_ref[...] * h
    for kk in range(C):
        o_ref[kk] = hp[:, kk * CW:(kk + 1) * CW]


def _mm(z, W, p2d):
    K = z.shape[1]
    return pl.pallas_call(
        _mm_body,
        grid=(N // BM,),
        in_specs=[
            pl.BlockSpec((BM, K), lambda m: (m, 0)),
            pl.BlockSpec((K, D), lambda m: (0, 0)),
            pl.BlockSpec((BM, 1), lambda m: (m, 0)),
        ],
        out_specs=pl.BlockSpec((C, BM, CW), lambda m: (0, m, 0)),
        out_shape=jax.ShapeDtypeStruct((C, N, CW), jnp.float32),
    )(z, W, p2d)


def _comb_body(decay, agg_ref, hp_ref, p_ref, c_ref, b_ref, a_ref, o_ref):
    pv = p_ref[...]
    cv = c_ref[...]
    for kk in range(C):
        u = (decay * pv * agg_ref[kk] + cv * hp_ref[kk]
             + b_ref[0, :, kk * CW:(kk + 1) * CW])
        o_ref[:, kk * CW:(kk + 1) * CW] = (
            jnp.maximum(u, 0.0)
            + a_ref[0, :, kk * CW:(kk + 1) * CW] * jnp.minimum(u, 0.0))


def _combine(aggc, hpc, p2d, c2d, b2, a2, decay):
    return pl.pallas_call(
        functools.partial(_comb_body, float(decay)),
        grid=(N // BM,),
        in_specs=[
            pl.BlockSpec((C, BM, CW), lambda m: (0, m, 0)),
            pl.BlockSpec((C, BM, CW), lambda m: (0, m, 0)),
            pl.BlockSpec((BM, 1), lambda m: (m, 0)),
            pl.BlockSpec((BM, 1), lambda m: (m, 0)),
            pl.BlockSpec((1, 1, D), lambda m: (0, 0, 0)),
            pl.BlockSpec((1, 1, D), lambda m: (0, 0, 0)),
        ],
        out_specs=pl.BlockSpec((BM, D), lambda m: (m, 0)),
        out_shape=jax.ShapeDtypeStruct((N, D), jnp.float32),
    )(aggc, hpc, p2d, c2d, b2, a2)


def kernel(x, edge_index, W0, b0, W1, b1, W2, b2, prelu_a):
    f32 = jnp.float32
    i32 = jnp.int32
    srcp = edge_index[0].astype(i32)
    dstp = edge_index[1].astype(i32)
    zeros_n = jnp.zeros((NP,), f32)
    zslab = jnp.zeros((NP, CW), f32)

    ewr, ews, pT, cT = _precompute(srcp, dstp, zeros_n)
    ewr = ewr.reshape(3, E0)
    ews = ews.reshape(3, NP)
    pT = pT.reshape(3, NP)
    cT = cT.reshape(3, NP)

    ew_T = jnp.concatenate([ewr[:, :E0], ews[:, :N]], axis=1)
    loop = jnp.arange(N, dtype=edge_index.dtype)
    ei = jnp.concatenate([edge_index, jnp.stack([loop, loop])], axis=1)
    ei_T = jnp.broadcast_to(ei[None], (3,) + ei.shape)

    params = [(W0, b0.reshape(1, 1, D)), (W1, b1.reshape(1, 1, D)),
              (W2, b2.reshape(1, 1, D))]
    a4 = prelu_a.reshape(1, 1, D)
    feats = []
    for t in range(3):
        p2d = pT[t, :N, None]
        c2d = cT[t, :N, None]
        z = x
        for (W, b4t) in params:
            hpc = _mm(z, W, p2d)
            aggc = _spmm(hpc, srcp, dstp, zslab)
            z = _combine(aggc, hpc, p2d, c2d, b4t, a4, DECAY[t])
        feats.append(z)
    features_T = jnp.stack(feats)
    return features_T, ei_T, ew_T


# speedup vs baseline: 9.4193x; 1.0004x over previous
import functools

import jax
import jax.numpy as jnp
from jax import lax
from jax.experimental import pallas as pl
from jax.experimental.pallas import tpu as pltpu
from jax.experimental.pallas import tpu_sc as plsc

N = 10000
NP = 10240
E0 = 160000
D = 512
C = 4
CW = 128
BM = 2000
EB = 80
TILE_E = E0 // 16
NBLK = TILE_E // EB
NPT = NP // 16
NPW = NP // 32
EPS = 0.001
DECAY = [(1.0 - EPS) ** t for t in range(3)]

_MESH = plsc.VectorSubcoreMesh(core_axis_name="c", subcore_axis_name="s",
                               num_cores=2, num_subcores=16)


def _rsqrt16(x):
    i = lax.bitcast_convert_type(x, jnp.int32)
    i = jnp.int32(0x5F3759DF) - (i >> 1)
    y = lax.bitcast_convert_type(i, jnp.float32)
    for _ in range(3):
        y = y * (1.5 - 0.5 * x * y * y)
    return y


def _pre_body(srcp, dstp, zn, ewr, ews, pT, cT,
              sfv, dfv, dloc, wn, qv, ones_v, degsl, qloc, ewt,
              pbuf, cbuf, ebuf, deg_s, q_s, dinv_s):
    cid = lax.axis_index("c")
    sid = lax.axis_index("s")
    ebase = sid * TILE_E
    nb640 = sid * NPT

    pltpu.sync_copy(srcp.at[pl.ds(ebase, TILE_E)], sfv)
    pltpu.sync_copy(dstp.at[pl.ds(ebase, TILE_E)], dfv)
    pltpu.sync_copy(zn.at[pl.ds(nb640, NPT)], deg_s.at[pl.ds(nb640, NPT)])
    pltpu.sync_copy(zn.at[pl.ds(nb640, NPT)], q_s.at[pl.ds(nb640, NPT)])
    def _fill1(i, _):
        ones_v[pl.ds(i * 16, 16)] = jnp.full((16,), 1.0, jnp.float32)
        return 0
    lax.fori_loop(0, TILE_E // 16, _fill1, 0)
    plsc.subcore_barrier()

    pltpu.sync_copy(ones_v, deg_s.at[dfv], add=True)
    plsc.subcore_barrier()

    pltpu.sync_copy(deg_s.at[pl.ds(nb640, NPT)], degsl)
    def _pa2(i, _):
        v = degsl[pl.ds(i * 16, 16)] + 1.0
        degsl[pl.ds(i * 16, 16)] = _rsqrt16(v)
        return 0
    lax.fori_loop(0, NPT // 16, _pa2, 0)
    pltpu.sync_copy(degsl, dinv_s.at[pl.ds(nb640, NPT)])
    plsc.subcore_barrier()
    pltpu.sync_copy(dinv_s, dloc)

    def _pb(i, _):
        off = i * 16
        s16 = sfv[pl.ds(off, 16)]
        d16 = dfv[pl.ds(off, 16)]
        sv = plsc.load_gather(dloc, [s16])
        dv = plsc.load_gather(dloc, [d16])
        qv[pl.ds(off, 16)] = sv
        wn[pl.ds(off, 16)] = sv * dv
        return 0
    lax.fori_loop(0, TILE_E // 16, _pb, 0)
    pltpu.sync_copy(qv, q_s.at[dfv], add=True)
    plsc.subcore_barrier()

    @pl.when(cid == 0)
    def _():
        for t in range(3):
            dk = jnp.float32(DECAY[t])
            def _sc(i, _, dk=dk):
                ewt[pl.ds(i * 16, 16)] = wn[pl.ds(i * 16, 16)] * dk
                return 0
            lax.fori_loop(0, TILE_E // 16, _sc, 0)
            pltpu.sync_copy(ewt, ewr.at[pl.ds(t * E0 + ebase, TILE_E)])

    w = cid * 16 + sid
    nb = w * NPW
    pltpu.sync_copy(q_s.at[pl.ds(nb, NPW)], qloc)
    for t in range(3):
        dk = jnp.float32(DECAY[t])
        omd = jnp.float32(1.0 - DECAY[t])
        def _pc(i, _, dk=dk, omd=omd):
            d0 = dloc[pl.ds(nb + i * 16, 16)]
            q16 = qloc[pl.ds(i * 16, 16)]
            sw = d0 * (q16 + d0)
            degt = dk * sw + omd
            dit = _rsqrt16(degt)
            p16 = dit * d0
            s16 = dit * dit * (dk * d0 * d0 + omd)
            pbuf[pl.ds(i * 16, 16)] = p16
            cbuf[pl.ds(i * 16, 16)] = s16 / p16
            ebuf[pl.ds(i * 16, 16)] = dk * d0 * d0 + omd
            return 0
        lax.fori_loop(0, NPW // 16, _pc, 0)
        pltpu.sync_copy(pbuf, pT.at[pl.ds(t * NP + nb, NPW)])
        pltpu.sync_copy(cbuf, cT.at[pl.ds(t * NP + nb, NPW)])
        pltpu.sync_copy(ebuf, ews.at[pl.ds(t * NP + nb, NPW)])


def _precompute(srcp, dstp, zeros_n):
    f32 = jnp.float32
    kfn = pl.kernel(
        _pre_body,
        out_type=(
            jax.ShapeDtypeStruct((3 * E0,), f32),
            jax.ShapeDtypeStruct((3 * NP,), f32),
            jax.ShapeDtypeStruct((3 * NP,), f32),
            jax.ShapeDtypeStruct((3 * NP,), f32),
        ),
        mesh=_MESH,
        scratch_types=[
            pltpu.VMEM((TILE_E,), jnp.int32),
            pltpu.VMEM((TILE_E,), jnp.int32),
            pltpu.VMEM((NP,), f32),
            pltpu.VMEM((TILE_E,), f32),
            pltpu.VMEM((TILE_E,), f32),
            pltpu.VMEM((TILE_E,), f32),
            pltpu.VMEM((NPT,), f32),
            pltpu.VMEM((NPW,), f32),
            pltpu.VMEM((TILE_E,), f32),
            pltpu.VMEM((NPW,), f32),
            pltpu.VMEM((NPW,), f32),
            pltpu.VMEM((NPW,), f32),
            pltpu.VMEM_SHARED((NP,), f32),
            pltpu.VMEM_SHARED((NP,), f32),
            pltpu.VMEM_SHARED((NP,), f32),
        ],
        compiler_params=pltpu.CompilerParams(needs_layout_passes=False),
    )
    return kfn(srcp, dstp, zeros_n)


def _spmm_body(hpc, srcp, dstp, zslab, aggc,
               sfv, d2v, r0, r1, sg0, sg1, ss0, ss1, slab):
    cid = lax.axis_index("c")
    sid = lax.axis_index("s")
    ebase = sid * TILE_E

    pltpu.sync_copy(srcp.at[pl.ds(ebase, TILE_E)], sfv)
    def _mkrows(j, _):
        pltpu.sync_copy(dstp.at[pl.ds(ebase + j * EB, EB)], d2v.at[j])
        return 0
    lax.fori_loop(0, NBLK, _mkrows, 0)

    for j in range(C // 2):
        cidx = cid * (C // 2) + j
        tbl = hpc.at[cidx]
        pltpu.sync_copy(zslab.at[pl.ds(sid * NPT, NPT)],
                        slab.at[pl.ds(sid * NPT, NPT)])
        plsc.subcore_barrier()

        pltpu.async_copy(tbl.at[sfv.at[pl.ds(0, EB)]], r0, sg0)
        pltpu.async_copy(tbl.at[sfv.at[pl.ds(EB, EB)]], r1, sg1)

        def _blk(k, _):
            j0 = 2 * k
            j1 = 2 * k + 1
            pltpu.make_async_copy(tbl.at[sfv.at[pl.ds(0, EB)]], r0, sg0).wait()
            pltpu.async_copy(r0, slab.at[d2v.at[j0]], ss0, add=True)
            pltpu.make_async_copy(tbl.at[sfv.at[pl.ds(0, EB)]], r1, sg1).wait()
            pltpu.async_copy(r1, slab.at[d2v.at[j1]], ss1, add=True)
            pltpu.make_async_copy(r0, slab.at[d2v.at[0]], ss0).wait()
            @pl.when(j0 + 2 < NBLK)
            def _():
                pltpu.async_copy(tbl.at[sfv.at[pl.ds((j0 + 2) * EB, EB)]],
                                 r0, sg0)
            pltpu.make_async_copy(r1, slab.at[d2v.at[0]], ss1).wait()
            @pl.when(j1 + 2 < NBLK)
            def _():
                pltpu.async_copy(tbl.at[sfv.at[pl.ds((j1 + 2) * EB, EB)]],
                                 r1, sg1)
            return 0
        lax.fori_loop(0, NBLK // 2, _blk, 0)
        if NBLK % 2 == 1:
            pltpu.make_async_copy(tbl.at[sfv.at[pl.ds(0, EB)]], r0, sg0).wait()
            pltpu.async_copy(r0, slab.at[d2v.at[NBLK - 1]], ss0, add=True)
            pltpu.make_async_copy(r0, slab.at[d2v.at[0]], ss0).wait()
        plsc.subcore_barrier()
        pltpu.sync_copy(slab.at[pl.ds(sid * NPT, NPT)],
                        aggc.at[cidx].at[pl.ds(sid * NPT, NPT)])
        plsc.subcore_barrier()


def _spmm(hpc, srcp, dstp, zslab):
    f32 = jnp.float32
    kfn = pl.kernel(
        _spmm_body,
        out_type=jax.ShapeDtypeStruct((C, NP, CW), f32),
        mesh=_MESH,
        scratch_types=[
            pltpu.VMEM((TILE_E,), jnp.int32),
            pltpu.VMEM((NBLK, EB), jnp.int32),
            pltpu.VMEM((EB, CW), f32),
            pltpu.VMEM((EB, CW), f32),
            pltpu.SemaphoreType.DMA,
            pltpu.SemaphoreType.DMA,
            pltpu.SemaphoreType.DMA,
            pltpu.SemaphoreType.DMA,
            pltpu.VMEM_SHARED((NP, CW), f32),
        ],
        compiler_params=pltpu.CompilerParams(needs_layout_passes=False),
    )
    return kfn(hpc, srcp, dstp, zslab)


def _mm_body(z_ref, w_ref, p_ref, o_ref):
    h = jnp.dot(z_ref[...], w_ref[...], preferred_element_type=jnp.float32)
    hp = p_ref[...] * h
    for kk in range(C):
        o_ref[kk] = hp[:, kk * CW:(kk + 1) * CW]


def _mm(z, W, p2d):
    K = z.shape[1]
    return pl.pallas_call(
        _mm_body,
        grid=(N // BM,),
        in_specs=[
            pl.BlockSpec((BM, K), lambda m: (m, 0)),
            pl.BlockSpec((K, D), lambda m: (0, 0)),
            pl.BlockSpec((BM, 1), lambda m: (m, 0)),
        ],
        out_specs=pl.BlockSpec((C, BM, CW), lambda m: (0, m, 0)),
        out_shape=jax.ShapeDtypeStruct((C, N, CW), jnp.float32),
    )(z, W, p2d)


def _comb_body(decay, agg_ref, hp_ref, p_ref, c_ref, b_ref, a_ref, o_ref):
    pv = p_ref[...]
    cv = c_ref[...]
    for kk in range(C):
        u = (decay * pv * agg_ref[kk] + cv * hp_ref[kk]
             + b_ref[0, :, kk * CW:(kk + 1) * CW])
        o_ref[:, kk * CW:(kk + 1) * CW] = (
            jnp.maximum(u, 0.0)
            + a_ref[0, :, kk * CW:(kk + 1) * CW] * jnp.minimum(u, 0.0))


def _combine(aggc, hpc, p2d, c2d, b2, a2, decay):
    return pl.pallas_call(
        functools.partial(_comb_body, float(decay)),
        grid=(N // BM,),
        in_specs=[
            pl.BlockSpec((C, BM, CW), lambda m: (0, m, 0)),
            pl.BlockSpec((C, BM, CW), lambda m: (0, m, 0)),
            pl.BlockSpec((BM, 1), lambda m: (m, 0)),
            pl.BlockSpec((BM, 1), lambda m: (m, 0)),
            pl.BlockSpec((1, 1, D), lambda m: (0, 0, 0)),
            pl.BlockSpec((1, 1, D), lambda m: (0, 0, 0)),
        ],
        out_specs=pl.BlockSpec((BM, D), lambda m: (m, 0)),
        out_shape=jax.ShapeDtypeStruct((N, D), jnp.float32),
    )(aggc, hpc, p2d, c2d, b2, a2)


def kernel(x, edge_index, W0, b0, W1, b1, W2, b2, prelu_a):
    f32 = jnp.float32
    i32 = jnp.int32
    srcp = edge_index[0].astype(i32)
    dstp = edge_index[1].astype(i32)
    zeros_n = jnp.zeros((NP,), f32)
    zslab = jnp.zeros((NP, CW), f32)

    ewr, ews, pT, cT = _precompute(srcp, dstp, zeros_n)
    ewr = ewr.reshape(3, E0)
    ews = ews.reshape(3, NP)
    pT = pT.reshape(3, NP)
    cT = cT.reshape(3, NP)

    ew_T = jnp.concatenate([ewr[:, :E0], ews[:, :N]], axis=1)
    loop = jnp.arange(N, dtype=edge_index.dtype)
    ei = jnp.concatenate([edge_index, jnp.stack([loop, loop])], axis=1)
    ei_T = jnp.broadcast_to(ei[None], (3,) + ei.shape)

    params = [(W0, b0.reshape(1, 1, D)), (W1, b1.reshape(1, 1, D)),
              (W2, b2.reshape(1, 1, D))]
    a4 = prelu_a.reshape(1, 1, D)
    feats = []
    for t in range(3):
        p2d = pT[t, :N, None]
        c2d = cT[t, :N, None]
        z = x
        for (W, b4t) in params:
            hpc = _mm(z, W, p2d)
            aggc = _spmm(hpc, srcp, dstp, zslab)
            z = _combine(aggc, hpc, p2d, c2d, b4t, a4, DECAY[t])
        feats.append(z)
    features_T = jnp.stack(feats)
    return features_T, ei_T, ew_T
